# Initial kernel scaffold; baseline (speedup 1.0000x reference)
import jax
import jax.numpy as jnp
from jax.experimental import pallas as pl


def kernel(indices, table, W1, b1, W2, b2):
    raise NotImplementedError("write your pallas kernel here")


# speedup vs baseline: 1.1600x; 1.1600x over previous
import functools

import jax
import jax.numpy as jnp
from jax import lax
from jax.experimental import pallas as pl
from jax.experimental.pallas import tpu as pltpu
from jax.experimental.pallas import tpu_sc as plsc

NC = 2
NS = 16
NW = NC * NS


def _make_sc_pool(B, L, D):
    rows_per_w = B // NW
    chunk = 100
    nchunk = L // chunk
    idx_rows_per_w = rows_per_w * nchunk
    inv_l = 1.0 / float(L)

    mesh = plsc.VectorSubcoreMesh(core_axis_name="c", subcore_axis_name="s")

    @functools.partial(
        pl.kernel,
        mesh=mesh,
        out_type=jax.ShapeDtypeStruct((B, D), jnp.float32),
        compiler_params=pltpu.CompilerParams(use_tc_tiling_on_sc=False),
        scratch_types=[
            pltpu.VMEM((idx_rows_per_w, chunk), jnp.int32),
            pltpu.VMEM((chunk, D), jnp.float32),
            pltpu.VMEM((chunk, D), jnp.float32),
            pltpu.VMEM((rows_per_w, D), jnp.float32),
            pltpu.SemaphoreType.DMA,
            pltpu.SemaphoreType.DMA,
        ],
    )
    def sc_pool(idx_hbm, table_hbm, out_hbm, idx_v, buf_a, buf_b, pooled_v,
                sem_a, sem_b):
        wid = lax.axis_index("s") * NC + lax.axis_index("c")
        idx_base = wid * idx_rows_per_w
        pltpu.sync_copy(idx_hbm.at[pl.ds(idx_base, idx_rows_per_w)], idx_v)

        def row_body(b, carry):
            cp_a = pltpu.async_copy(table_hbm.at[idx_v.at[2 * b]], buf_a, sem_a)
            cp_b = pltpu.async_copy(table_hbm.at[idx_v.at[2 * b + 1]], buf_b,
                                    sem_b)
            cp_a.wait()
            cp_b.wait()

            def red(j, acc):
                return acc + buf_a[j] + buf_b[j]

            acc = lax.fori_loop(0, chunk, red, jnp.zeros((D,), jnp.float32))
            pooled_v[b] = acc * inv_l
            return carry

        lax.fori_loop(0, rows_per_w, row_body, 0)
        pltpu.sync_copy(pooled_v, out_hbm.at[pl.ds(wid * rows_per_w, rows_per_w)])

    return sc_pool


def _dense_body(pooled_ref, w1_ref, b1_ref, w2_ref, b2_ref, out_ref):
    p = pooled_ref[...]
    h = jnp.maximum(
        jnp.dot(p, w1_ref[...], preferred_element_type=jnp.float32)
        + b1_ref[...], 0.0)
    z = jnp.dot(h, w2_ref[...], preferred_element_type=jnp.float32) + b2_ref[...]
    out_ref[...] = 1.0 / (1.0 + jnp.exp(-z))


@jax.jit
def kernel(indices, table, W1, b1, W2, b2):
    B, L = indices.shape
    D = table.shape[1]
    n_class = W2.shape[1]

    idx2 = indices.astype(jnp.int32).reshape(B * 2, L // 2)
    pooled = _make_sc_pool(B, L, D)(idx2, table)

    out = pl.pallas_call(
        _dense_body,
        out_shape=jax.ShapeDtypeStruct((B, n_class), jnp.float32),
    )(pooled, W1, b1.reshape(1, D), W2, b2.reshape(1, n_class))
    return out


# speedup vs baseline: 1.3373x; 1.1529x over previous
import functools

import jax
import jax.numpy as jnp
from jax import lax
from jax.experimental import pallas as pl
from jax.experimental.pallas import tpu as pltpu
from jax.experimental.pallas import tpu_sc as plsc

NC = 2
NS = 16
NW = NC * NS
CH = 128
NBUF = 4


def _make_sc_pool(B, L, D):
    rows_per_w = B // NW
    flat_per_w = rows_per_w * L
    nchunk = flat_per_w // CH

    mesh = plsc.VectorSubcoreMesh(core_axis_name="c", subcore_axis_name="s")

    @functools.partial(
        pl.kernel,
        mesh=mesh,
        out_type=jax.ShapeDtypeStruct((B, D), jnp.float32),
        compiler_params=pltpu.CompilerParams(use_tc_tiling_on_sc=False),
        scratch_types=[
            pltpu.VMEM((nchunk, CH), jnp.int32),
            pltpu.VMEM((nchunk, CH), jnp.int32),
            pltpu.VMEM((rows_per_w, D), jnp.float32),
            pltpu.VMEM_SHARED((NS * rows_per_w, D), jnp.float32),
        ]
        + [pltpu.VMEM((CH, D), jnp.float32) for _ in range(NBUF)]
        + [pltpu.SemaphoreType.DMA for _ in range(NBUF)],
    )
    def sc_pool(idx_hbm, didx_hbm, table_hbm, out_hbm, idx_v, didx_v,
                zeros_v, pooled_sh, *bufs_and_sems):
        bufs = bufs_and_sems[:NBUF]
        sems = bufs_and_sems[NBUF:]
        sid = lax.axis_index("s")
        wid = sid * NC + lax.axis_index("c")
        pltpu.sync_copy(idx_hbm.at[pl.ds(wid * nchunk, nchunk)], idx_v)
        pltpu.sync_copy(didx_hbm.at[pl.ds(sid * nchunk, nchunk)], didx_v)

        def zero_body(i, carry):
            zeros_v[i] = jnp.zeros((D,), jnp.float32)
            return carry

        lax.fori_loop(0, rows_per_w, zero_body, 0)
        pltpu.sync_copy(zeros_v, pooled_sh.at[pl.ds(sid * rows_per_w,
                                                    rows_per_w)])

        for b in range(NBUF):
            pltpu.make_async_copy(
                table_hbm.at[idx_v.at[b]], bufs[b], sems[b]).start()

        def group_body(g, carry):
            for b in range(NBUF):
                t = g * NBUF + b
                pltpu.make_async_copy(
                    table_hbm.at[idx_v.at[t]], bufs[b], sems[b]).wait()
                pltpu.sync_copy(bufs[b], pooled_sh.at[didx_v.at[t]], add=True)
                pltpu.make_async_copy(
                    table_hbm.at[idx_v.at[t + NBUF]], bufs[b], sems[b]).start()
            return carry

        lax.fori_loop(0, nchunk // NBUF - 1, group_body, 0)

        for b in range(NBUF):
            t = nchunk - NBUF + b
            pltpu.make_async_copy(
                table_hbm.at[idx_v.at[t]], bufs[b], sems[b]).wait()
            pltpu.sync_copy(bufs[b], pooled_sh.at[didx_v.at[t]], add=True)

        pltpu.sync_copy(pooled_sh.at[pl.ds(sid * rows_per_w, rows_per_w)],
                        out_hbm.at[pl.ds(wid * rows_per_w, rows_per_w)])

    return sc_pool


def _dense_body(pooled_ref, w1_ref, b1_ref, w2_ref, b2_ref, out_ref):
    p = pooled_ref[...]
    h = jnp.maximum(
        jnp.dot(p, w1_ref[...], preferred_element_type=jnp.float32)
        + b1_ref[...], 0.0)
    z = jnp.dot(h, w2_ref[...], preferred_element_type=jnp.float32) + b2_ref[...]
    out_ref[...] = 1.0 / (1.0 + jnp.exp(-z))


@jax.jit
def kernel(indices, table, W1, b1, W2, b2):
    B, L = indices.shape
    D = table.shape[1]
    n_class = W2.shape[1]
    flat_per_w = (B // NW) * L
    nchunk = flat_per_w // CH

    idx2 = indices.astype(jnp.int32).reshape(B * L // CH, CH)
    local = jnp.arange(flat_per_w, dtype=jnp.int32) // L
    didx = (local[None, :] + (jnp.arange(NS, dtype=jnp.int32)
                              * (B // NW))[:, None]).reshape(NS * nchunk, CH)

    sums = _make_sc_pool(B, L, D)(idx2, didx, table)

    out = pl.pallas_call(
        _dense_body,
        out_shape=jax.ShapeDtypeStruct((B, n_class), jnp.float32),
    )(sums, W1 * (1.0 / L), b1.reshape(1, D), W2, b2.reshape(1, n_class))
    return out


# speedup vs baseline: 1.3857x; 1.0362x over previous
import functools

import jax
import jax.numpy as jnp
from jax import lax
from jax.experimental import pallas as pl
from jax.experimental.pallas import tpu as pltpu
from jax.experimental.pallas import tpu_sc as plsc

NC = 2
NS = 16
NW = NC * NS
CH = 100
RING = 4
NACC = 4


def _make_sc_pool(B, L, D):
    rows_per_w = B // NW
    nchunk = L // CH
    idx_rows_per_w = rows_per_w * nchunk

    mesh = plsc.VectorSubcoreMesh(core_axis_name="c", subcore_axis_name="s")

    @functools.partial(
        pl.kernel,
        mesh=mesh,
        out_type=jax.ShapeDtypeStruct((B, D), jnp.float32),
        compiler_params=pltpu.CompilerParams(use_tc_tiling_on_sc=False),
        scratch_types=[
            pltpu.VMEM((idx_rows_per_w, CH), jnp.int32),
            pltpu.VMEM((rows_per_w, D), jnp.float32),
        ]
        + [pltpu.VMEM((L, D), jnp.float32) for _ in range(RING)]
        + [pltpu.SemaphoreType.DMA for _ in range(RING)],
    )
    def sc_pool(idx_hbm, table_hbm, out_hbm, idx_v, pooled_v, *bufs_and_sems):
        bufs = bufs_and_sems[:RING]
        sems = bufs_and_sems[RING:]
        wid = lax.axis_index("s") * NC + lax.axis_index("c")
        pltpu.sync_copy(idx_hbm.at[pl.ds(wid * idx_rows_per_w,
                                         idx_rows_per_w)], idx_v)

        def start_row(row, slot):
            for c in range(nchunk):
                pltpu.make_async_copy(
                    table_hbm.at[idx_v.at[nchunk * row + c]],
                    bufs[slot].at[pl.ds(c * CH, CH)],
                    sems[slot]).start()

        def finish_row(row, slot):
            for c in range(nchunk):
                pltpu.make_async_copy(
                    table_hbm.at[idx_v.at[nchunk * row + c]],
                    bufs[slot].at[pl.ds(c * CH, CH)],
                    sems[slot]).wait()
            buf = bufs[slot]
            accs = [buf[a] for a in range(NACC)]
            for j in range(NACC, L):
                accs[j % NACC] = accs[j % NACC] + buf[j]
            total = (accs[0] + accs[1]) + (accs[2] + accs[3])
            pooled_v[row] = total

        for r in range(RING):
            start_row(r, r)

        def group_body(g, carry):
            for r in range(RING):
                row = g * RING + r
                finish_row(row, r)
                start_row(row + RING, r)
            return carry

        lax.fori_loop(0, rows_per_w // RING - 1, group_body, 0)

        for r in range(RING):
            finish_row(rows_per_w - RING + r, r)

        pltpu.sync_copy(pooled_v, out_hbm.at[pl.ds(wid * rows_per_w,
                                                   rows_per_w)])

    return sc_pool


def _dense_body(pooled_ref, w1_ref, b1_ref, w2_ref, b2_ref, out_ref):
    p = pooled_ref[...]
    h = jnp.maximum(
        jnp.dot(p, w1_ref[...], preferred_element_type=jnp.float32)
        + b1_ref[...], 0.0)
    z = jnp.dot(h, w2_ref[...], preferred_element_type=jnp.float32) + b2_ref[...]
    out_ref[...] = 1.0 / (1.0 + jnp.exp(-z))


@jax.jit
def kernel(indices, table, W1, b1, W2, b2):
    B, L = indices.shape
    D = table.shape[1]
    n_class = W2.shape[1]

    idx2 = indices.astype(jnp.int32).reshape(B * L // CH, CH)
    sums = _make_sc_pool(B, L, D)(idx2, table)

    out = pl.pallas_call(
        _dense_body,
        out_shape=jax.ShapeDtypeStruct((B, n_class), jnp.float32),
    )(sums, W1 * (1.0 / L), b1.reshape(1, D), W2, b2.reshape(1, n_class))
    return out
